# Initial kernel scaffold; baseline (speedup 1.0000x reference)
import jax
import jax.numpy as jnp
from jax.experimental import pallas as pl


def kernel(loss):
    raise NotImplementedError("write your pallas kernel here")


# speedup vs baseline: 14.9509x; 14.9509x over previous
import functools

import jax
import jax.numpy as jnp
import numpy as np
from jax import lax
from jax.experimental import pallas as pl
from jax.experimental.pallas import tpu as pltpu
from jax.experimental.pallas import tpu_sc as plsc

_N = 1048576
_ALPHA = 0.05
_K = int(np.searchsorted((np.arange(_N) / _N).astype(np.float32),
                         np.float32(1.0 - _ALPHA), side="left"))

_NT = 16
_CHUNK = _N // _NT
_VPC = _CHUNK // 16
_HB = 256 * 16
_FB = 16384

_mesh = plsc.VectorSubcoreMesh(core_axis_name="c", subcore_axis_name="s")


@functools.partial(
    pl.kernel,
    mesh=_mesh,
    compiler_params=pltpu.CompilerParams(needs_layout_passes=False),
    out_type=jax.ShapeDtypeStruct((16,), jnp.float32),
    scratch_types=[
        pltpu.VMEM((_CHUNK,), jnp.int32),
        pltpu.VMEM((_FB,), jnp.float32),
        pltpu.VMEM((_HB,), jnp.int32),
        pltpu.VMEM((_HB,), jnp.int32),
        pltpu.VMEM((512,), jnp.float32),
        pltpu.VMEM((16,), jnp.float32),
        pltpu.VMEM_SHARED((_NT, _HB), jnp.int32),
        pltpu.VMEM_SHARED((512,), jnp.float32),
    ],
)
def _cvar_sc(bits_hbm, loss_hbm, out_hbm, bits_v, fbuf_v, hist_v, tmp_v,
             red_v, out_v, hist_sh, red_sh):
    core = lax.axis_index("c")
    sid = lax.axis_index("s")

    lane = lax.iota(jnp.int32, 16)
    ones = jnp.full((16,), 1, jnp.int32)
    zeros_i = jnp.full((16,), 0, jnp.int32)

    pltpu.sync_copy(bits_hbm.at[pl.ds(sid * _CHUNK, _CHUNK)], bits_v)

    def keys_at(i):
        b = bits_v[pl.ds(i * 16, 16)]
        return b ^ (lax.shift_right_arithmetic(b, 31) & 0x7FFFFFFF)

    prefix = jnp.int32(0)
    rank = jnp.int32(_K)

    for rnd in range(4):
        shift = 24 - 8 * rnd

        def zero_body(i, _):
            hist_v[pl.ds(i * 16, 16)] = zeros_i
            return 0
        lax.fori_loop(0, _HB // 16, zero_body, 0)

        if rnd == 0:
            def hist_body(i, _):
                key = keys_at(i)
                buck = (lax.shift_right_arithmetic(key, 24) & 0xFF) ^ 0x80
                idx = (buck << 4) | lane
                plsc.addupdate_scatter(hist_v, [idx], ones)
                return 0
        else:
            hi_sh = shift + 8
            prefhi = lax.broadcast(
                lax.shift_right_arithmetic(prefix, hi_sh), (16,))

            def hist_body(i, _):
                key = keys_at(i)
                m = lax.shift_right_arithmetic(key, hi_sh) == prefhi
                buck = lax.shift_right_arithmetic(key, shift) & 0xFF
                idx = (buck << 4) | lane
                plsc.addupdate_scatter(hist_v, [idx], ones, mask=m)
                return 0
        lax.fori_loop(0, _VPC, hist_body, 0)

        pltpu.sync_copy(hist_v, hist_sh.at[sid])
        plsc.subcore_barrier()
        for step in (8, 4, 2, 1):
            @pl.when(sid < step)
            def _():
                pltpu.sync_copy(hist_sh.at[sid + step], tmp_v)

                def add_body(i, _):
                    s = pl.ds(i * 16, 16)
                    hist_v[s] = hist_v[s] + tmp_v[s]
                    return 0
                lax.fori_loop(0, _HB // 16, add_body, 0)
                pltpu.sync_copy(hist_v, hist_sh.at[sid])
            plsc.subcore_barrier()
        pltpu.sync_copy(hist_sh.at[0], hist_v)
        plsc.subcore_barrier()

        def scan_body(b, carry):
            cum, selb, cumbef = carry
            row = hist_v[pl.ds(b * 16, 16)]
            tot = jnp.sum(row)
            newcum = cum + tot
            hit = jnp.logical_and(selb < 0, newcum > rank)
            selb = lax.select(hit, b, selb)
            cumbef = lax.select(hit, cum, cumbef)
            return newcum, selb, cumbef

        _, selb, cumbef = lax.fori_loop(
            0, 256, scan_body,
            (jnp.int32(0), jnp.int32(-1), jnp.int32(0)))
        digit = selb ^ 0x80 if rnd == 0 else selb
        prefix = prefix | (digit << shift)
        rank = rank - cumbef

    thresh = lax.select(prefix == 0, jnp.int32(-1), prefix)
    thresh_vec = lax.broadcast(thresh, (16,))

    acc = jnp.full((16,), 0.0, jnp.float32)
    cnt = jnp.full((16,), 0.0, jnp.float32)
    for blk in range(_CHUNK // _FB):
        pltpu.sync_copy(
            loss_hbm.at[pl.ds(sid * _CHUNK + blk * _FB, _FB)], fbuf_v)

        def tail_body(i, carry):
            a, c = carry
            m = keys_at(blk * (_FB // 16) + i) >= thresh_vec
            f = fbuf_v[pl.ds(i * 16, 16)]
            a = a + jnp.where(m, f, 0.0)
            c = c + jnp.where(m, 1.0, 0.0)
            return a, c

        acc, cnt = lax.fori_loop(0, _FB // 16, tail_body, (acc, cnt))

    red_v[pl.ds(0, 16)] = acc
    red_v[pl.ds(16, 16)] = cnt
    pltpu.sync_copy(red_v.at[pl.ds(0, 32)], red_sh.at[pl.ds(sid * 32, 32)])
    plsc.subcore_barrier()
    pltpu.sync_copy(red_sh, red_v)

    def red_body(t, carry):
        a, c = carry
        a = a + red_v[pl.ds(t * 32, 16)]
        c = c + red_v[pl.ds(t * 32 + 16, 16)]
        return a, c

    acc, cnt = lax.fori_loop(
        0, _NT, red_body,
        (jnp.full((16,), 0.0, jnp.float32), jnp.full((16,), 0.0, jnp.float32)))
    s_vec = lax.broadcast(jnp.sum(acc), (16,))
    c_vec = lax.broadcast(jnp.sum(cnt), (16,))
    out_v[pl.ds(0, 16)] = s_vec / c_vec

    @pl.when(jnp.logical_and(core == 0, sid == 0))
    def _():
        pltpu.sync_copy(out_v, out_hbm)


@jax.jit
def kernel(loss):
    bits = lax.bitcast_convert_type(loss, jnp.int32)
    return _cvar_sc(bits, loss)[0]


# speedup vs baseline: 17.6746x; 1.1822x over previous
import functools

import jax
import jax.numpy as jnp
import numpy as np
from jax import lax
from jax.experimental import pallas as pl
from jax.experimental.pallas import tpu as pltpu
from jax.experimental.pallas import tpu_sc as plsc

_N = 1048576
_ALPHA = 0.05
_K = int(np.searchsorted((np.arange(_N) / _N).astype(np.float32),
                         np.float32(1.0 - _ALPHA), side="left"))

_NT = 16
_CHUNK = _N // _NT
_VPC = _CHUNK // 16
_HB = 256 * 16
_FB = 16384

_mesh = plsc.VectorSubcoreMesh(core_axis_name="c", subcore_axis_name="s")


@functools.partial(
    pl.kernel,
    mesh=_mesh,
    compiler_params=pltpu.CompilerParams(needs_layout_passes=False),
    out_type=jax.ShapeDtypeStruct((16,), jnp.float32),
    scratch_types=[
        pltpu.VMEM((_CHUNK,), jnp.int32),
        pltpu.VMEM((_FB,), jnp.float32),
        pltpu.VMEM((_HB,), jnp.int32),
        pltpu.VMEM((_HB,), jnp.int32),
        pltpu.VMEM((512,), jnp.float32),
        pltpu.VMEM((16,), jnp.float32),
        pltpu.VMEM_SHARED((_NT, _HB), jnp.int32),
        pltpu.VMEM_SHARED((512,), jnp.float32),
    ],
)
def _cvar_sc(bits_hbm, loss_hbm, out_hbm, bits_v, fbuf_v, hist_v, tmp_v,
             red_v, out_v, hist_sh, red_sh):
    core = lax.axis_index("c")
    sid = lax.axis_index("s")

    lane = lax.iota(jnp.int32, 16)
    ones = jnp.full((16,), 1, jnp.int32)
    zeros_i = jnp.full((16,), 0, jnp.int32)

    pltpu.sync_copy(bits_hbm.at[pl.ds(sid * _CHUNK, _CHUNK)], bits_v)

    def keys_at(i):
        b = bits_v[pl.ds(i * 16, 16)]
        return b ^ (lax.shift_right_arithmetic(b, 31) & 0x7FFFFFFF)

    prefix = jnp.int32(0)
    rank = jnp.int32(_K)

    for rnd in range(4):
        shift = 24 - 8 * rnd

        def zero_body(i, _):
            for u in range(8):
                hist_v[pl.ds((i * 8 + u) * 16, 16)] = zeros_i
            return 0
        lax.fori_loop(0, _HB // 128, zero_body, 0)

        if rnd == 0:
            def hist_body(i, _):
                for u in range(8):
                    key = keys_at(i * 8 + u)
                    buck = (lax.shift_right_arithmetic(key, 24) & 0xFF) ^ 0x80
                    idx = (buck << 4) | lane
                    plsc.addupdate_scatter(hist_v, [idx], ones)
                return 0
        else:
            hi_sh = shift + 8
            prefhi = lax.broadcast(
                lax.shift_right_arithmetic(prefix, hi_sh), (16,))

            def hist_body(i, _):
                for u in range(8):
                    key = keys_at(i * 8 + u)
                    m = lax.shift_right_arithmetic(key, hi_sh) == prefhi
                    buck = lax.shift_right_arithmetic(key, shift) & 0xFF
                    idx = (buck << 4) | lane
                    plsc.addupdate_scatter(hist_v, [idx], ones, mask=m)
                return 0
        lax.fori_loop(0, _VPC // 8, hist_body, 0)

        pltpu.sync_copy(hist_v, hist_sh.at[sid])
        plsc.subcore_barrier()
        for step in (8, 4, 2, 1):
            @pl.when(sid < step)
            def _():
                pltpu.sync_copy(hist_sh.at[sid + step], tmp_v)

                def add_body(i, _):
                    for u in range(8):
                        s = pl.ds((i * 8 + u) * 16, 16)
                        hist_v[s] = hist_v[s] + tmp_v[s]
                    return 0
                lax.fori_loop(0, _HB // 128, add_body, 0)
                pltpu.sync_copy(hist_v, hist_sh.at[sid])
            plsc.subcore_barrier()
        pltpu.sync_copy(hist_sh.at[0], hist_v)
        plsc.subcore_barrier()

        def scan_body(b, carry):
            cum, selb, cumbef = carry
            row = hist_v[pl.ds(b * 16, 16)]
            tot = jnp.sum(row)
            newcum = cum + tot
            hit = jnp.logical_and(selb < 0, newcum > rank)
            selb = lax.select(hit, b, selb)
            cumbef = lax.select(hit, cum, cumbef)
            return newcum, selb, cumbef

        _, selb, cumbef = lax.fori_loop(
            0, 256, scan_body,
            (jnp.int32(0), jnp.int32(-1), jnp.int32(0)))
        digit = selb ^ 0x80 if rnd == 0 else selb
        prefix = prefix | (digit << shift)
        rank = rank - cumbef

    thresh = lax.select(prefix == 0, jnp.int32(-1), prefix)
    thresh_vec = lax.broadcast(thresh, (16,))

    accs = [jnp.full((16,), 0.0, jnp.float32) for _ in range(4)]
    cnts = [jnp.full((16,), 0.0, jnp.float32) for _ in range(4)]
    for blk in range(_CHUNK // _FB):
        pltpu.sync_copy(
            loss_hbm.at[pl.ds(sid * _CHUNK + blk * _FB, _FB)], fbuf_v)

        def tail_body(i, carry):
            aa = list(carry[0])
            cc = list(carry[1])
            for u in range(8):
                j = i * 8 + u
                m = keys_at(blk * (_FB // 16) + j) >= thresh_vec
                f = fbuf_v[pl.ds(j * 16, 16)]
                aa[u % 4] = aa[u % 4] + jnp.where(m, f, 0.0)
                cc[u % 4] = cc[u % 4] + jnp.where(m, 1.0, 0.0)
            return tuple(aa), tuple(cc)

        accs, cnts = lax.fori_loop(
            0, _FB // 128, tail_body, (tuple(accs), tuple(cnts)))
    acc = (accs[0] + accs[1]) + (accs[2] + accs[3])
    cnt = (cnts[0] + cnts[1]) + (cnts[2] + cnts[3])

    red_v[pl.ds(0, 16)] = acc
    red_v[pl.ds(16, 16)] = cnt
    pltpu.sync_copy(red_v.at[pl.ds(0, 32)], red_sh.at[pl.ds(sid * 32, 32)])
    plsc.subcore_barrier()
    pltpu.sync_copy(red_sh, red_v)

    def red_body(t, carry):
        a, c = carry
        a = a + red_v[pl.ds(t * 32, 16)]
        c = c + red_v[pl.ds(t * 32 + 16, 16)]
        return a, c

    acc, cnt = lax.fori_loop(
        0, _NT, red_body,
        (jnp.full((16,), 0.0, jnp.float32), jnp.full((16,), 0.0, jnp.float32)))
    s_vec = lax.broadcast(jnp.sum(acc), (16,))
    c_vec = lax.broadcast(jnp.sum(cnt), (16,))
    out_v[pl.ds(0, 16)] = s_vec / c_vec

    @pl.when(jnp.logical_and(core == 0, sid == 0))
    def _():
        pltpu.sync_copy(out_v, out_hbm)


@jax.jit
def kernel(loss):
    bits = lax.bitcast_convert_type(loss, jnp.int32)
    return _cvar_sc(bits, loss)[0]
